# Initial kernel scaffold; baseline (speedup 1.0000x reference)
import jax
import jax.numpy as jnp
from jax.experimental import pallas as pl


def kernel(x, conv_neigh_indices, down_neigh_indices, down_indices, W1, b1, gamma1, beta1, W2, b2, gamma2, beta2):
    raise NotImplementedError("write your pallas kernel here")


# speedup vs baseline: 8.6575x; 8.6575x over previous
import functools

import jax
import jax.numpy as jnp
from jax import lax
from jax.experimental import pallas as pl
from jax.experimental.pallas import tpu as pltpu
from jax.experimental.pallas import tpu_sc as plsc

V = 655362
VD = 163842
NB = 2
C = 16
R = NB * C
K = 7

NC, NS = 2, 16
NW = NC * NS
CH = 128
NCH = 41
T = CH * NCH
VDP = NW * T

_EPS = 1e-5
_SLOPE = 0.2


def _to_rows(x2d):
    blk = 1024
    grid = pl.cdiv(V, blk)

    def body(x_ref, o_ref):
        o_ref[...] = x_ref[...].T

    return pl.pallas_call(
        body,
        grid=(grid,),
        in_specs=[pl.BlockSpec((R, blk), lambda i: (0, i))],
        out_specs=pl.BlockSpec((blk, R), lambda i: (i, 0)),
        out_shape=jax.ShapeDtypeStruct((V, R), jnp.float32),
    )(x2d)


def _gather_add7(table, idx3, n_table):
    mesh = plsc.VectorSubcoreMesh(
        core_axis_name="c", subcore_axis_name="s",
        num_cores=NC, num_subcores=NS)

    @functools.partial(
        pl.kernel,
        out_type=jax.ShapeDtypeStruct((VDP, R), jnp.float32),
        mesh=mesh,
        scratch_types=[
            pltpu.VMEM((K, CH), jnp.int32),
            pltpu.VMEM((CH, R), jnp.float32),
            pltpu.SemaphoreType.DMA,
            pltpu.SemaphoreType.DMA,
        ],
        compiler_params=pltpu.CompilerParams(use_tc_tiling_on_sc=False),
    )
    def run(table_ref, idx_ref, out_ref, idx_v, acc_v, sem0, sem1):
        wid = lax.axis_index("s") * NC + lax.axis_index("c")

        @pl.loop(0, NCH)
        def _chunk(ci):
            start = (wid * NCH + ci) * CH
            pltpu.sync_copy(idx_ref.at[wid * NCH + ci], idx_v)
            pltpu.async_copy(table_ref.at[idx_v.at[0]], acc_v, sem0).wait()
            cps = [
                pltpu.async_copy(
                    table_ref.at[idx_v.at[j]], acc_v, sem1, add=True)
                for j in range(1, K)
            ]
            for cp in cps:
                cp.wait()
            pltpu.sync_copy(acc_v, out_ref.at[pl.ds(start, CH)])

    return run(table, idx3)


def _matmul7(inp, bd, scale=None, shift=None):
    blk = 2048
    grid = VDP // blk
    apply_aff = scale is not None

    def body(*refs):
        if apply_aff:
            x_ref, bd_ref, s_ref, t_ref, o_ref = refs
            xv = x_ref[...] * s_ref[...] + t_ref[...]
            xv = jnp.where(xv >= 0, xv, _SLOPE * xv)
        else:
            x_ref, bd_ref, o_ref = refs
            xv = x_ref[...]
        res = jnp.dot(xv, bd_ref[...], preferred_element_type=jnp.float32)
        for k in range(K):
            o_ref[k, :, :] = res[:, k * R:(k + 1) * R]

    in_specs = [
        pl.BlockSpec((blk, R), lambda i: (i, 0)),
        pl.BlockSpec((R, K * R), lambda i: (0, 0)),
    ]
    args = [inp, bd]
    if apply_aff:
        in_specs += [
            pl.BlockSpec((1, R), lambda i: (0, 0)),
            pl.BlockSpec((1, R), lambda i: (0, 0)),
        ]
        args += [scale, shift]
    return pl.pallas_call(
        body,
        grid=(grid,),
        in_specs=in_specs,
        out_specs=pl.BlockSpec((K, blk, R), lambda i: (0, i, 0)),
        out_shape=jax.ShapeDtypeStruct((K, VDP, R), jnp.float32),
    )(*args)


def _stats(h):
    blk = 4096
    grid = VDP // blk

    def body(h_ref, o_ref):
        i = pl.program_id(0)
        xv = h_ref[...]
        rows = lax.broadcasted_iota(jnp.int32, (blk, R), 0) + i * blk
        xm = jnp.where(rows < VD, xv, 0.0)
        s = jnp.sum(xm, axis=0, keepdims=True)
        sq = jnp.sum(xm * xm, axis=0, keepdims=True)
        part = jnp.concatenate([s, sq], axis=0)

        @pl.when(i == 0)
        def _():
            o_ref[...] = part

        @pl.when(i != 0)
        def _():
            o_ref[...] += part

    return pl.pallas_call(
        body,
        grid=(grid,),
        in_specs=[pl.BlockSpec((blk, R), lambda i: (i, 0))],
        out_specs=pl.BlockSpec((2, R), lambda i: (0, 0)),
        out_shape=jax.ShapeDtypeStruct((2, R), jnp.float32),
    )(h)


def _finalize(h, scale, shift):
    blk = 1024
    grid = pl.cdiv(VD, blk)

    def body(h_ref, s_ref, t_ref, o_ref):
        z = h_ref[...] * s_ref[...] + t_ref[...]
        z = jnp.where(z >= 0, z, _SLOPE * z)
        o_ref[...] = z.T

    return pl.pallas_call(
        body,
        grid=(grid,),
        in_specs=[
            pl.BlockSpec((blk, R), lambda i: (i, 0)),
            pl.BlockSpec((1, R), lambda i: (0, 0)),
            pl.BlockSpec((1, R), lambda i: (0, 0)),
        ],
        out_specs=pl.BlockSpec((R, blk), lambda i: (0, i)),
        out_shape=jax.ShapeDtypeStruct((R, VD), jnp.float32),
    )(h, scale, shift)


def _block_diag_w(W):
    A = jnp.transpose(W.reshape(C, C, K), (1, 0, 2))
    eye = jnp.eye(NB, dtype=W.dtype)
    bd = jnp.einsum('ab,iok->aikbo', eye, A)
    return bd.reshape(R, K * R)


def _affine_params(st, gamma, beta):
    n = float(NB * VD)
    s2 = st[0].reshape(NB, C).sum(axis=0)
    q2 = st[1].reshape(NB, C).sum(axis=0)
    mean = s2 / n
    var = q2 / n - mean * mean
    sc = gamma / jnp.sqrt(var + _EPS)
    sh = beta - mean * sc
    return (jnp.tile(sc, NB)[None, :], jnp.tile(sh, NB)[None, :])


def _chunk_idx(idxT):
    return jnp.transpose(idxT.reshape(K, NW * NCH, CH), (1, 0, 2))


def kernel(x, conv_neigh_indices, down_neigh_indices, down_indices,
           W1, b1, gamma1, beta1, W2, b2, gamma2, beta2):
    del down_indices, b1, b2
    x2d = x.reshape(R, V)
    xT = _to_rows(x2d)

    dnT = jnp.zeros((K, VDP), jnp.int32).at[:, :VD].set(
        down_neigh_indices.T)
    offs = (jnp.arange(K, dtype=jnp.int32) * VDP)[:, None]
    cnF = jnp.zeros((K, VDP), jnp.int32).at[:, :VD].set(
        conv_neigh_indices.T + offs)
    dn3 = _chunk_idx(dnT)
    cn3 = _chunk_idx(cnF)

    xp = _gather_add7(xT, dn3, V)
    bd1 = _block_diag_w(W1) * (1.0 / K)
    y1 = _matmul7(xp, bd1)
    h1 = _gather_add7(y1.reshape(K * VDP, R), cn3, K * VDP)
    sc1, sh1 = _affine_params(_stats(h1), gamma1, beta1)

    bd2 = _block_diag_w(W2)
    y2 = _matmul7(h1, bd2, sc1, sh1)
    h2 = _gather_add7(y2.reshape(K * VDP, R), cn3, K * VDP)
    sc2, sh2 = _affine_params(_stats(h2), gamma2, beta2)

    out2d = _finalize(h2, sc2, sh2)
    return out2d.reshape(NB, C, VD), None


# speedup vs baseline: 8.8278x; 1.0197x over previous
import functools

import jax
import jax.numpy as jnp
from jax import lax
from jax.experimental import pallas as pl
from jax.experimental.pallas import tpu as pltpu
from jax.experimental.pallas import tpu_sc as plsc

V = 655362
VD = 163842
NB = 2
C = 16
R = NB * C
K = 7

NC, NS = 2, 16
NW = NC * NS
CH = 656
NCH = 8
T = CH * NCH
VDP = NW * T

_EPS = 1e-5
_SLOPE = 0.2


def _to_rows(x2d):
    blk = 1024
    grid = pl.cdiv(V, blk)

    def body(x_ref, o_ref):
        o_ref[...] = x_ref[...].T

    return pl.pallas_call(
        body,
        grid=(grid,),
        in_specs=[pl.BlockSpec((R, blk), lambda i: (0, i))],
        out_specs=pl.BlockSpec((blk, R), lambda i: (i, 0)),
        out_shape=jax.ShapeDtypeStruct((V, R), jnp.float32),
    )(x2d)


def _gather_add7(table, idx3, n_table):
    mesh = plsc.VectorSubcoreMesh(
        core_axis_name="c", subcore_axis_name="s",
        num_cores=NC, num_subcores=NS)

    @functools.partial(
        pl.kernel,
        out_type=jax.ShapeDtypeStruct((VDP, R), jnp.float32),
        mesh=mesh,
        scratch_types=[
            pltpu.VMEM((K, CH), jnp.int32),
            pltpu.VMEM((K, CH), jnp.int32),
            pltpu.VMEM((CH, R), jnp.float32),
            pltpu.VMEM((CH, R), jnp.float32),
            pltpu.SemaphoreType.DMA,
            pltpu.SemaphoreType.DMA,
            pltpu.SemaphoreType.DMA,
            pltpu.SemaphoreType.DMA,
            pltpu.SemaphoreType.DMA,
            pltpu.SemaphoreType.DMA,
            pltpu.SemaphoreType.DMA,
            pltpu.SemaphoreType.DMA,
        ],
        compiler_params=pltpu.CompilerParams(use_tc_tiling_on_sc=False),
    )
    def run(table_ref, idx_ref, out_ref, idx_a, idx_b, acc_a, acc_b,
            si_a, si_b, sb_a, sb_b, sg_a, sg_b, sw_a, sw_b):
        wid = lax.axis_index("s") * NC + lax.axis_index("c")
        idxv, accv = (idx_a, idx_b), (acc_a, acc_b)
        semi, semb = (si_a, si_b), (sb_a, sb_b)
        semg, semw = (sg_a, sg_b), (sw_a, sw_b)

        cp_i = [None, None]
        pend_adds = [None, None]
        pend_wb = [None, None]
        cp_i[0] = pltpu.async_copy(idx_ref.at[wid * NCH], idx_a, si_a)
        for ci in range(NCH):
            p = ci & 1
            q = 1 - p
            if pend_wb[p] is not None:
                pend_wb[p].wait()
            cp_i[p].wait()
            cb = pltpu.async_copy(table_ref.at[idxv[p].at[0]], accv[p],
                                  semb[p])
            if pend_adds[q] is not None:
                for cp in pend_adds[q]:
                    cp.wait()
                st_q = (wid * NCH + ci - 1) * CH
                pend_wb[q] = pltpu.async_copy(
                    accv[q], out_ref.at[pl.ds(st_q, CH)], semw[q])
            if ci + 1 < NCH:
                cp_i[q] = pltpu.async_copy(
                    idx_ref.at[wid * NCH + ci + 1], idxv[q], semi[q])
            cb.wait()
            pend_adds[p] = [
                pltpu.async_copy(
                    table_ref.at[idxv[p].at[j]], accv[p], semg[p], add=True)
                for j in range(1, K)
            ]
        pL = (NCH - 1) & 1
        for cp in pend_adds[pL]:
            cp.wait()
        st_l = (wid * NCH + NCH - 1) * CH
        pend_wb[pL] = pltpu.async_copy(
            accv[pL], out_ref.at[pl.ds(st_l, CH)], semw[pL])
        for t in (0, 1):
            if pend_wb[t] is not None:
                pend_wb[t].wait()

    return run(table, idx3)


def _matmul7(inp, bd, scale=None, shift=None):
    blk = 2048
    grid = VDP // blk
    apply_aff = scale is not None

    def body(*refs):
        if apply_aff:
            x_ref, bd_ref, s_ref, t_ref, o_ref = refs
            xv = x_ref[...] * s_ref[...] + t_ref[...]
            xv = jnp.where(xv >= 0, xv, _SLOPE * xv)
        else:
            x_ref, bd_ref, o_ref = refs
            xv = x_ref[...]
        res = jnp.dot(xv, bd_ref[...], preferred_element_type=jnp.float32)
        for k in range(K):
            o_ref[k, :, :] = res[:, k * R:(k + 1) * R]

    in_specs = [
        pl.BlockSpec((blk, R), lambda i: (i, 0)),
        pl.BlockSpec((R, K * R), lambda i: (0, 0)),
    ]
    args = [inp, bd]
    if apply_aff:
        in_specs += [
            pl.BlockSpec((1, R), lambda i: (0, 0)),
            pl.BlockSpec((1, R), lambda i: (0, 0)),
        ]
        args += [scale, shift]
    return pl.pallas_call(
        body,
        grid=(grid,),
        in_specs=in_specs,
        out_specs=pl.BlockSpec((K, blk, R), lambda i: (0, i, 0)),
        out_shape=jax.ShapeDtypeStruct((K, VDP, R), jnp.float32),
    )(*args)


def _stats(h):
    blk = 4096
    grid = VDP // blk

    def body(h_ref, o_ref):
        i = pl.program_id(0)
        xv = h_ref[...]
        rows = lax.broadcasted_iota(jnp.int32, (blk, R), 0) + i * blk
        xm = jnp.where(rows < VD, xv, 0.0)
        s = jnp.sum(xm, axis=0, keepdims=True)
        sq = jnp.sum(xm * xm, axis=0, keepdims=True)
        part = jnp.concatenate([s, sq], axis=0)

        @pl.when(i == 0)
        def _():
            o_ref[...] = part

        @pl.when(i != 0)
        def _():
            o_ref[...] += part

    return pl.pallas_call(
        body,
        grid=(grid,),
        in_specs=[pl.BlockSpec((blk, R), lambda i: (i, 0))],
        out_specs=pl.BlockSpec((2, R), lambda i: (0, 0)),
        out_shape=jax.ShapeDtypeStruct((2, R), jnp.float32),
    )(h)


def _finalize(h, scale, shift):
    blk = 1024
    grid = pl.cdiv(VD, blk)

    def body(h_ref, s_ref, t_ref, o_ref):
        z = h_ref[...] * s_ref[...] + t_ref[...]
        z = jnp.where(z >= 0, z, _SLOPE * z)
        o_ref[...] = z.T

    return pl.pallas_call(
        body,
        grid=(grid,),
        in_specs=[
            pl.BlockSpec((blk, R), lambda i: (i, 0)),
            pl.BlockSpec((1, R), lambda i: (0, 0)),
            pl.BlockSpec((1, R), lambda i: (0, 0)),
        ],
        out_specs=pl.BlockSpec((R, blk), lambda i: (0, i)),
        out_shape=jax.ShapeDtypeStruct((R, VD), jnp.float32),
    )(h, scale, shift)


def _block_diag_w(W):
    A = jnp.transpose(W.reshape(C, C, K), (1, 0, 2))
    eye = jnp.eye(NB, dtype=W.dtype)
    bd = jnp.einsum('ab,iok->aikbo', eye, A)
    return bd.reshape(R, K * R)


def _affine_params(st, gamma, beta):
    n = float(NB * VD)
    s2 = st[0].reshape(NB, C).sum(axis=0)
    q2 = st[1].reshape(NB, C).sum(axis=0)
    mean = s2 / n
    var = q2 / n - mean * mean
    sc = gamma / jnp.sqrt(var + _EPS)
    sh = beta - mean * sc
    return (jnp.tile(sc, NB)[None, :], jnp.tile(sh, NB)[None, :])


def _chunk_idx(idxT):
    return jnp.transpose(idxT.reshape(K, NW * NCH, CH), (1, 0, 2))


def kernel(x, conv_neigh_indices, down_neigh_indices, down_indices,
           W1, b1, gamma1, beta1, W2, b2, gamma2, beta2):
    del down_indices, b1, b2
    x2d = x.reshape(R, V)
    xT = _to_rows(x2d)

    dnT = jnp.zeros((K, VDP), jnp.int32).at[:, :VD].set(
        down_neigh_indices.T)
    offs = (jnp.arange(K, dtype=jnp.int32) * VDP)[:, None]
    cnF = jnp.zeros((K, VDP), jnp.int32).at[:, :VD].set(
        conv_neigh_indices.T + offs)
    dn3 = _chunk_idx(dnT)
    cn3 = _chunk_idx(cnF)

    xp = _gather_add7(xT, dn3, V)
    bd1 = _block_diag_w(W1) * (1.0 / K)
    y1 = _matmul7(xp, bd1)
    h1 = _gather_add7(y1.reshape(K * VDP, R), cn3, K * VDP)
    sc1, sh1 = _affine_params(_stats(h1), gamma1, beta1)

    bd2 = _block_diag_w(W2)
    y2 = _matmul7(h1, bd2, sc1, sh1)
    h2 = _gather_add7(y2.reshape(K * VDP, R), cn3, K * VDP)
    sc2, sh2 = _affine_params(_stats(h2), gamma2, beta2)

    out2d = _finalize(h2, sc2, sh2)
    return out2d.reshape(NB, C, VD), None


# speedup vs baseline: 15.5122x; 1.7572x over previous
import functools

import jax
import jax.numpy as jnp
from jax import lax
from jax.experimental import pallas as pl
from jax.experimental.pallas import tpu as pltpu
from jax.experimental.pallas import tpu_sc as plsc

V = 655362
VD = 163842
NB = 2
C = 16
R = NB * C
K = 7

NC, NS = 2, 16
NW = NC * NS
CH = 656
NCH = 8
VDP = NW * NCH * CH

_EPS = 1e-5
_SLOPE = 0.2


VDP4 = VDP // 4


def _gather_add7(table, idx3, n_table):
    mesh = plsc.VectorSubcoreMesh(
        core_axis_name="c", subcore_axis_name="s",
        num_cores=NC, num_subcores=NS)

    @functools.partial(
        pl.kernel,
        out_type=jax.ShapeDtypeStruct((VDP, R), jnp.float32),
        mesh=mesh,
        scratch_types=[
            pltpu.VMEM((K, CH), jnp.int32),
            pltpu.VMEM((K, CH), jnp.int32),
            pltpu.VMEM((CH, R), jnp.float32),
            pltpu.VMEM((CH, R), jnp.float32),
            pltpu.SemaphoreType.DMA,
            pltpu.SemaphoreType.DMA,
            pltpu.SemaphoreType.DMA,
            pltpu.SemaphoreType.DMA,
            pltpu.SemaphoreType.DMA,
            pltpu.SemaphoreType.DMA,
            pltpu.SemaphoreType.DMA,
            pltpu.SemaphoreType.DMA,
        ],
        compiler_params=pltpu.CompilerParams(use_tc_tiling_on_sc=False),
    )
    def run(table_ref, idx_ref, out_ref, idx_a, idx_b, acc_a, acc_b,
            si_a, si_b, sb_a, sb_b, sg_a, sg_b, sw_a, sw_b):
        wid = lax.axis_index("s") * NC + lax.axis_index("c")
        idxv, accv = (idx_a, idx_b), (acc_a, acc_b)
        semi, semb = (si_a, si_b), (sb_a, sb_b)
        semg, semw = (sg_a, sg_b), (sw_a, sw_b)

        cp_i = [None, None]
        pend_adds = [None, None]
        pend_wb = [None, None]
        cp_i[0] = pltpu.async_copy(idx_ref.at[wid * NCH], idx_a, si_a)
        for ci in range(NCH):
            p = ci & 1
            q = 1 - p
            if pend_wb[p] is not None:
                pend_wb[p].wait()
            cp_i[p].wait()
            cb = pltpu.async_copy(table_ref.at[idxv[p].at[0]], accv[p],
                                  semb[p])
            if pend_adds[q] is not None:
                for cp in pend_adds[q]:
                    cp.wait()
                st_q = (wid * NCH + ci - 1) * CH
                pend_wb[q] = pltpu.async_copy(
                    accv[q], out_ref.at[pl.ds(st_q, CH)], semw[q])
            if ci + 1 < NCH:
                cp_i[q] = pltpu.async_copy(
                    idx_ref.at[wid * NCH + ci + 1], idxv[q], semi[q])
            cb.wait()
            pend_adds[p] = [
                pltpu.async_copy(
                    table_ref.at[idxv[p].at[j]], accv[p], semg[p], add=True)
                for j in range(1, K)
            ]
        pL = (NCH - 1) & 1
        for cp in pend_adds[pL]:
            cp.wait()
        st_l = (wid * NCH + NCH - 1) * CH
        pend_wb[pL] = pltpu.async_copy(
            accv[pL], out_ref.at[pl.ds(st_l, CH)], semw[pL])
        for t in (0, 1):
            if pend_wb[t] is not None:
                pend_wb[t].wait()

    return run(table, idx3)


def _matmul7(inp4, bd4, scale=None, shift=None):
    blk = 512
    grid = VDP4 // blk
    apply_aff = scale is not None

    def body(*refs):
        if apply_aff:
            x_ref, bd_ref, s_ref, t_ref, o_ref = refs
            xv = x_ref[...] * s_ref[...] + t_ref[...]
            xv = jnp.where(xv >= 0, xv, _SLOPE * xv)
        else:
            x_ref, bd_ref, o_ref = refs
            xv = x_ref[...]
        res = jnp.dot(xv, bd_ref[...], preferred_element_type=jnp.float32)
        for k in range(K):
            o_ref[k, :, :] = res[:, k * 4 * R:(k + 1) * 4 * R]

    in_specs = [
        pl.BlockSpec((blk, 4 * R), lambda i: (i, 0)),
        pl.BlockSpec((4 * R, K * 4 * R), lambda i: (0, 0)),
    ]
    args = [inp4, bd4]
    if apply_aff:
        in_specs += [
            pl.BlockSpec((1, 4 * R), lambda i: (0, 0)),
            pl.BlockSpec((1, 4 * R), lambda i: (0, 0)),
        ]
        args += [scale, shift]
    return pl.pallas_call(
        body,
        grid=(grid,),
        in_specs=in_specs,
        out_specs=pl.BlockSpec((K, blk, 4 * R), lambda i: (0, i, 0)),
        out_shape=jax.ShapeDtypeStruct((K, VDP4, 4 * R), jnp.float32),
    )(*args)


def _stats(h4):
    blk = 1024
    grid = VDP4 // blk

    def body(h_ref, o_ref):
        i = pl.program_id(0)
        xv = h_ref[...]
        rows = lax.broadcasted_iota(jnp.int32, (blk, 4 * R), 0) + i * blk
        lane = lax.broadcasted_iota(jnp.int32, (blk, 4 * R), 1)
        vert = rows * 4 + lane // R
        xm = jnp.where(vert < VD, xv, 0.0)
        s = jnp.sum(xm, axis=0, keepdims=True)
        sq = jnp.sum(xm * xm, axis=0, keepdims=True)
        part = jnp.concatenate([s, sq], axis=0)

        @pl.when(i == 0)
        def _():
            o_ref[...] = part

        @pl.when(i != 0)
        def _():
            o_ref[...] += part

    return pl.pallas_call(
        body,
        grid=(grid,),
        in_specs=[pl.BlockSpec((blk, 4 * R), lambda i: (i, 0))],
        out_specs=pl.BlockSpec((2, 4 * R), lambda i: (0, 0)),
        out_shape=jax.ShapeDtypeStruct((2, 4 * R), jnp.float32),
    )(h4)


def _finalize(h, scale, shift):
    blk = 1024
    grid = pl.cdiv(VD, blk)

    def body(h_ref, s_ref, t_ref, o_ref):
        z = h_ref[...] * s_ref[...] + t_ref[...]
        z = jnp.where(z >= 0, z, _SLOPE * z)
        o_ref[...] = z.T

    return pl.pallas_call(
        body,
        grid=(grid,),
        in_specs=[
            pl.BlockSpec((blk, R), lambda i: (i, 0)),
            pl.BlockSpec((1, R), lambda i: (0, 0)),
            pl.BlockSpec((1, R), lambda i: (0, 0)),
        ],
        out_specs=pl.BlockSpec((R, blk), lambda i: (0, i)),
        out_shape=jax.ShapeDtypeStruct((R, VD), jnp.float32),
    )(h, scale, shift)


def _block_diag_w(W):
    A = jnp.transpose(W.reshape(C, C, K), (1, 0, 2))
    eye = jnp.eye(NB, dtype=W.dtype)
    bd = jnp.einsum('ab,iok->aikbo', eye, A)
    return bd.reshape(R, K * R)


def _bd4(bd):
    A = bd.reshape(R, K, R)
    eye4 = jnp.eye(4, dtype=bd.dtype)
    out = jnp.einsum('qp,rks->qrkps', eye4, A)
    return out.reshape(4 * R, K * 4 * R)


def _affine_params(st, gamma, beta):
    n = float(NB * VD)
    s2 = st[0].reshape(4, NB, C).sum(axis=(0, 1))
    q2 = st[1].reshape(4, NB, C).sum(axis=(0, 1))
    mean = s2 / n
    var = q2 / n - mean * mean
    sc = gamma / jnp.sqrt(var + _EPS)
    sh = beta - mean * sc
    return (jnp.tile(sc, NB)[None, :], jnp.tile(sh, NB)[None, :])


def _chunk_idx(idxT):
    return jnp.transpose(idxT.reshape(K, NW * NCH, CH), (1, 0, 2))


def kernel(x, conv_neigh_indices, down_neigh_indices, down_indices,
           W1, b1, gamma1, beta1, W2, b2, gamma2, beta2):
    del down_indices, b1, b2
    x2d = x.reshape(R, V)
    xT = jnp.transpose(x2d)

    dnT = jnp.zeros((K, VDP), jnp.int32).at[:, :VD].set(
        down_neigh_indices.T)
    offs = (jnp.arange(K, dtype=jnp.int32) * VDP)[:, None]
    cnF = jnp.zeros((K, VDP), jnp.int32).at[:, :VD].set(
        conv_neigh_indices.T + offs)
    dn3 = _chunk_idx(dnT)
    cn3 = _chunk_idx(cnF)

    xp = _gather_add7(xT, dn3, V)
    bd1 = _bd4(_block_diag_w(W1) * (1.0 / K))
    y1 = _matmul7(xp.reshape(VDP4, 4 * R), bd1)
    h1 = _gather_add7(y1.reshape(K * VDP, R), cn3, K * VDP)
    h14 = h1.reshape(VDP4, 4 * R)
    sc1, sh1 = _affine_params(_stats(h14), gamma1, beta1)

    bd2 = _bd4(_block_diag_w(W2))
    y2 = _matmul7(h14, bd2, jnp.tile(sc1, (1, 4)),
                  jnp.tile(sh1, (1, 4)))
    h2 = _gather_add7(y2.reshape(K * VDP, R), cn3, K * VDP)
    sc2, sh2 = _affine_params(_stats(h2.reshape(VDP4, 4 * R)),
                              gamma2, beta2)

    out2d = _finalize(h2, sc2, sh2)
    return out2d.reshape(NB, C, VD), None
